# Initial kernel scaffold; baseline (speedup 1.0000x reference)
import functools
import math

import jax
import jax.numpy as jnp
from jax import lax
from jax.experimental import pallas as pl
from jax.experimental.pallas import tpu as pltpu
from jax.experimental.pallas import tpu_sc as plsc

EMB = 64
SCALE = 8.0
LANES = 16
CHUNK = 640


@functools.lru_cache(maxsize=None)
def _build(n_idx: int, vocab: int):
    info = plsc.get_sparse_core_info()
    nc, ns = info.num_cores, info.num_subcores
    nw = nc * ns
    per_w = n_idx // nw
    assert n_idx % nw == 0 and per_w % CHUNK == 0
    n_chunks = per_w // CHUNK

    mesh = plsc.VectorSubcoreMesh(core_axis_name="c", subcore_axis_name="s")

    @functools.partial(
        pl.kernel,
        mesh=mesh,
        out_type=jax.ShapeDtypeStruct((n_idx, EMB), jnp.float32),
        scratch_types=[
            pltpu.VMEM((per_w,), jnp.int32),
            pltpu.VMEM((CHUNK, EMB), jnp.float32),
            pltpu.SemaphoreType.DMA,
        ],
    )
    def gather_scale(table_hbm, idx_hbm, out_hbm, idx_v, rows_v, sem):
        wid = lax.axis_index("s") * nc + lax.axis_index("c")
        base = wid * per_w
        pltpu.sync_copy(idx_hbm.at[pl.ds(base, per_w)], idx_v)

        def chunk_body(c, _):
            off = c * CHUNK
            pltpu.async_copy(
                table_hbm.at[idx_v.at[pl.ds(off, CHUNK)]], rows_v, sem
            ).wait()

            def row_body(i, _):
                for j in range(EMB // LANES):
                    sl = pl.ds(j * LANES, LANES)
                    rows_v[i, sl] = rows_v[i, sl] * SCALE
                return 0

            lax.fori_loop(0, CHUNK, row_body, 0)
            pltpu.sync_copy(rows_v, out_hbm.at[pl.ds(base + off, CHUNK)])
            return 0

        lax.fori_loop(0, n_chunks, chunk_body, 0)

    return gather_scale


def kernel(src, SenEmbedding_dict, embedding_weight):
    l, b = src.shape
    vocab, emb = embedding_weight.shape
    idx = src.reshape(-1).astype(jnp.int32)
    fn = _build(l * b, vocab)
    out = fn(embedding_weight, idx)
    return out.reshape(l, b, emb)


# speedup vs baseline: 2.9444x; 2.9444x over previous
import functools
import math

import jax
import jax.numpy as jnp
from jax import lax
from jax.experimental import pallas as pl
from jax.experimental.pallas import tpu as pltpu
from jax.experimental.pallas import tpu_sc as plsc

EMB = 64
SCALE = 8.0
LANES = 16
CHUNK = 640


@functools.lru_cache(maxsize=None)
def _build(n_idx: int, vocab: int):
    info = plsc.get_sparse_core_info()
    nc, ns = info.num_cores, info.num_subcores
    nw = nc * ns
    per_w = n_idx // nw
    assert n_idx % nw == 0 and per_w % CHUNK == 0
    n_chunks = per_w // CHUNK

    mesh = plsc.VectorSubcoreMesh(core_axis_name="c", subcore_axis_name="s")

    @functools.partial(
        pl.kernel,
        mesh=mesh,
        compiler_params=pltpu.CompilerParams(use_tc_tiling_on_sc=False),
        out_type=jax.ShapeDtypeStruct((n_idx, EMB), jnp.float32),
        scratch_types=[
            pltpu.VMEM((per_w,), jnp.int32),
            pltpu.VMEM((CHUNK, EMB), jnp.float32),
            pltpu.SemaphoreType.DMA,
        ],
    )
    def gather_scale(table_hbm, idx_hbm, out_hbm, idx_v, rows_v, sem):
        wid = lax.axis_index("s") * nc + lax.axis_index("c")
        base = wid * per_w
        pltpu.sync_copy(idx_hbm.at[pl.ds(base, per_w)], idx_v)

        def chunk_body(c, _):
            off = c * CHUNK
            pltpu.async_copy(
                table_hbm.at[idx_v.at[pl.ds(off, CHUNK)]], rows_v, sem
            ).wait()

            def row_body(i, _):
                for j in range(EMB // LANES):
                    sl = pl.ds(j * LANES, LANES)
                    rows_v[i, sl] = rows_v[i, sl] * SCALE
                return 0

            lax.fori_loop(0, CHUNK, row_body, 0)
            pltpu.sync_copy(rows_v, out_hbm.at[pl.ds(base + off, CHUNK)])
            return 0

        lax.fori_loop(0, n_chunks, chunk_body, 0)

    return gather_scale


def kernel(src, SenEmbedding_dict, embedding_weight):
    l, b = src.shape
    vocab, emb = embedding_weight.shape
    idx = src.reshape(-1).astype(jnp.int32)
    fn = _build(l * b, vocab)
    out = fn(embedding_weight, idx)
    return out.reshape(l, b, emb)


# speedup vs baseline: 3.3559x; 1.1398x over previous
import functools

import jax
import jax.numpy as jnp
from jax import lax
from jax.experimental import pallas as pl
from jax.experimental.pallas import tpu as pltpu
from jax.experimental.pallas import tpu_sc as plsc

EMB = 64
SCALE = 8.0
LANES = 16
CHUNK = 400
ROW_UNROLL = 8


@functools.lru_cache(maxsize=None)
def _build(n_idx: int, vocab: int):
    info = plsc.get_sparse_core_info()
    nc, ns = info.num_cores, info.num_subcores
    nw = nc * ns
    per_w = n_idx // nw
    assert n_idx % nw == 0 and per_w % (2 * CHUNK) == 0
    n_chunks = per_w // CHUNK

    mesh = plsc.VectorSubcoreMesh(core_axis_name="c", subcore_axis_name="s")

    @functools.partial(
        pl.kernel,
        mesh=mesh,
        compiler_params=pltpu.CompilerParams(use_tc_tiling_on_sc=False),
        out_type=jax.ShapeDtypeStruct((n_idx, EMB), jnp.float32),
        scratch_types=[
            pltpu.VMEM((per_w,), jnp.int32),
            pltpu.VMEM((CHUNK, EMB), jnp.float32),
            pltpu.VMEM((CHUNK, EMB), jnp.float32),
            pltpu.VMEM((CHUNK, EMB), jnp.float32),
            pltpu.VMEM((CHUNK, EMB), jnp.float32),
            pltpu.SemaphoreType.DMA,
            pltpu.SemaphoreType.DMA,
            pltpu.SemaphoreType.DMA,
            pltpu.SemaphoreType.DMA,
        ],
    )
    def gather_scale(
        table_hbm, idx_hbm, out_hbm,
        idx_v, in0, in1, ou0, ou1, g0, g1, s0, s1,
    ):
        ins = (in0, in1)
        ous = (ou0, ou1)
        gsems = (g0, g1)
        ssems = (s0, s1)
        wid = lax.axis_index("s") * nc + lax.axis_index("c")
        base = wid * per_w
        pltpu.sync_copy(idx_hbm.at[pl.ds(base, per_w)], idx_v)

        def gather_start(c, b):
            pltpu.async_copy(
                table_hbm.at[idx_v.at[pl.ds(c * CHUNK, CHUNK)]],
                ins[b], gsems[b],
            )

        gather_start(0, 0)
        gather_start(1, 1)

        def outer(cc, _):
            for b in range(2):
                c = cc + b
                pltpu.make_async_copy(
                    table_hbm.at[idx_v.at[pl.ds(c * CHUNK, CHUNK)]],
                    ins[b], gsems[b],
                ).wait()
                @pl.when(cc >= 2)
                def _():
                    pltpu.make_async_copy(
                        ous[b], out_hbm.at[pl.ds(base, CHUNK)], ssems[b]
                    ).wait()

                def scale_body(i, _):
                    for r in range(ROW_UNROLL):
                        row = i * ROW_UNROLL + r
                        for j in range(EMB // LANES):
                            sl = pl.ds(j * LANES, LANES)
                            ous[b][row, sl] = ins[b][row, sl] * SCALE
                    return 0

                lax.fori_loop(0, CHUNK // ROW_UNROLL, scale_body, 0)

                @pl.when(cc < n_chunks - 2)
                def _():
                    gather_start(c + 2, b)

                pltpu.async_copy(
                    ous[b], out_hbm.at[pl.ds(base + c * CHUNK, CHUNK)], ssems[b]
                )
            return 0

        lax.fori_loop(0, n_chunks // 2, lambda t, x: outer(t * 2, x), 0)

        for b in range(2):
            pltpu.make_async_copy(
                ous[b], out_hbm.at[pl.ds(base, CHUNK)], ssems[b]
            ).wait()

    return gather_scale


def kernel(src, SenEmbedding_dict, embedding_weight):
    l, b = src.shape
    vocab, emb = embedding_weight.shape
    idx = src.reshape(-1).astype(jnp.int32)
    fn = _build(l * b, vocab)
    out = fn(embedding_weight, idx)
    return out.reshape(l, b, emb)
